# Initial kernel scaffold; baseline (speedup 1.0000x reference)
import jax
import jax.numpy as jnp
from jax.experimental import pallas as pl


def kernel(x, codebooks):
    raise NotImplementedError("write your pallas kernel here")


# speedup vs baseline: 1.2709x; 1.2709x over previous
import functools

import jax
import jax.numpy as jnp
from jax import lax
from jax.experimental import pallas as pl
from jax.experimental.pallas import tpu as pltpu
from jax.experimental.pallas import tpu_sc as plsc

B, S, DIM = 8, 576, 256
N = B * S
K = 8192
L = 8
COMMIT_W = 0.25

N_TILE = 512
K_TILE = 1024
NT = N // N_TILE
KT = K // K_TILE


def _mm_default(r, cb):
    return lax.dot_general(r, cb, (((1,), (1,)), ((), ())),
                           precision=lax.Precision.DEFAULT,
                           preferred_element_type=jnp.float32)


def _mm_bf16(r, cb):
    return lax.dot_general(r.astype(jnp.bfloat16), cb.astype(jnp.bfloat16),
                           (((1,), (1,)), ((), ())),
                           preferred_element_type=jnp.float32)


def _mm_highest(r, cb):
    return lax.dot_general(r, cb, (((1,), (1,)), ((), ())),
                           precision=lax.Precision.HIGHEST,
                           preferred_element_type=jnp.float32)


_MM = _mm_default


def _dist_argmin_body(has_q, *refs):
    if has_q:
        (r_ref, q_ref, cb_ref, b2_ref, res_ref, idx_ref, sse_ref,
         amin_ref, aidx_ref, a2_ref) = refs
    else:
        (r_ref, cb_ref, b2_ref, res_ref, idx_ref, sse_ref,
         amin_ref, aidx_ref, a2_ref) = refs
    n = pl.program_id(0)
    k = pl.program_id(1)

    @pl.when(k == 0)
    def _init():
        r = r_ref[...]
        if has_q:
            r = r - q_ref[...]
        res_ref[...] = r
        rsq = r * r
        a2_ref[...] = jnp.sum(rsq, axis=1, keepdims=True)
        part = jnp.sum(rsq)[None, None]
        sse_ref[...] = jnp.where(n == 0, part, sse_ref[...] + part)
        amin_ref[...] = jnp.full(amin_ref.shape, jnp.inf, jnp.float32)
        aidx_ref[...] = jnp.zeros(aidx_ref.shape, jnp.int32)

    r = res_ref[...]
    cb = cb_ref[...]
    t = _MM(r, cb)
    d2 = jnp.maximum((a2_ref[...] + b2_ref[...]) - 2.0 * t, 0.0)
    tmin = jnp.min(d2, axis=1, keepdims=True)
    ii = lax.broadcasted_iota(jnp.int32, d2.shape, 1)
    tidx = jnp.min(jnp.where(d2 == tmin, ii, jnp.int32(K_TILE)),
                   axis=1, keepdims=True)
    better = tmin < amin_ref[...]
    aidx_ref[...] = jnp.where(better, tidx + k * K_TILE, aidx_ref[...])
    amin_ref[...] = jnp.where(better, tmin, amin_ref[...])

    @pl.when(k == KT - 1)
    def _fin():
        idx_ref[...] = aidx_ref[...]


def _make_level_call(has_q):
    in_specs = [pl.BlockSpec((N_TILE, DIM), lambda n, k: (n, 0))]
    if has_q:
        in_specs.append(pl.BlockSpec((N_TILE, DIM), lambda n, k: (n, 0)))
    in_specs += [
        pl.BlockSpec((K_TILE, DIM), lambda n, k: (k, 0)),
        pl.BlockSpec((1, K_TILE), lambda n, k: (0, k)),
    ]
    return pl.pallas_call(
        functools.partial(_dist_argmin_body, has_q),
        grid=(NT, KT),
        in_specs=in_specs,
        out_specs=[
            pl.BlockSpec((N_TILE, DIM), lambda n, k: (n, 0)),
            pl.BlockSpec((N_TILE, 1), lambda n, k: (n, 0)),
            pl.BlockSpec((1, 1), lambda n, k: (0, 0)),
        ],
        out_shape=[
            jax.ShapeDtypeStruct((N, DIM), jnp.float32),
            jax.ShapeDtypeStruct((N, 1), jnp.int32),
            jax.ShapeDtypeStruct((1, 1), jnp.float32),
        ],
        scratch_shapes=[
            pltpu.VMEM((N_TILE, 1), jnp.float32),
            pltpu.VMEM((N_TILE, 1), jnp.int32),
            pltpu.VMEM((N_TILE, 1), jnp.float32),
        ],
        compiler_params=pltpu.CompilerParams(
            dimension_semantics=("arbitrary", "arbitrary")),
    )


_level_first = _make_level_call(False)
_level_next = _make_level_call(True)


def _b2_body(cb_ref, out_ref):
    cb = cb_ref[0]
    cbsq = cb * cb
    ones = jnp.ones((1, DIM), jnp.float32)
    out_ref[0] = lax.dot_general(ones, cbsq, (((1,), (1,)), ((), ())),
                                 precision=lax.Precision.HIGHEST,
                                 preferred_element_type=jnp.float32)


_b2_call = pl.pallas_call(
    _b2_body,
    grid=(L,),
    in_specs=[pl.BlockSpec((1, K, DIM), lambda l: (l, 0, 0))],
    out_specs=pl.BlockSpec((1, 1, K), lambda l: (l, 0, 0)),
    out_shape=jax.ShapeDtypeStruct((L, 1, K), jnp.float32),
    compiler_params=pltpu.CompilerParams(
        dimension_semantics=("arbitrary",)),
)


_NC, _NS = 2, 16
_NW = _NC * _NS
_BPW = N // _NW
_HALF = _BPW // 2
_KPC = K // _NS


def _sc_gather_count(cb_hbm, idx_hbm, out_hbm, counts_hbm,
                     idx_v, rows_v, stage_v, ones_v, cnt_sh, sem):
    c = lax.axis_index("c")
    s = lax.axis_index("s")
    wid = s * _NC + c
    base = wid * _BPW
    for j in range(2):
        pltpu.sync_copy(idx_hbm.at[pl.ds(base + j * _HALF, _HALF)],
                        idx_v.at[j])
    cps = [pltpu.async_copy(cb_hbm.at[idx_v.at[j]],
                            rows_v.at[pl.ds(j * _HALF, _HALF)], sem)
           for j in range(2)]
    for cp in cps:
        cp.wait()
    pltpu.sync_copy(rows_v, out_hbm.at[pl.ds(base, _BPW)])

    def _zero(i, _):
        stage_v[pl.ds(i * 16, 16)] = jnp.zeros((16,), jnp.float32)
        return 0

    lax.fori_loop(0, _KPC // 16, _zero, 0)

    def _one(i, _):
        ones_v[pl.ds(i * 16, 16)] = jnp.ones((16,), jnp.float32)
        return 0

    lax.fori_loop(0, (_HALF + 15) // 16, _one, 0)

    pltpu.sync_copy(stage_v, cnt_sh.at[pl.ds(s * _KPC, _KPC)])
    plsc.subcore_barrier()
    for j in range(2):
        pltpu.sync_copy(ones_v.at[pl.ds(0, _HALF)],
                        cnt_sh.at[idx_v.at[j]], add=True)
    plsc.subcore_barrier()
    pltpu.sync_copy(cnt_sh.at[pl.ds(s * _KPC, _KPC)], stage_v)
    pltpu.sync_copy(stage_v, counts_hbm.at[pl.ds(c * K + s * _KPC, _KPC)])


@functools.cache
def _get_gather_count():
    return pl.kernel(
        _sc_gather_count,
        mesh=plsc.VectorSubcoreMesh(core_axis_name="c",
                                    subcore_axis_name="s"),
        out_type=[
            jax.ShapeDtypeStruct((N, DIM), jnp.float32),
            jax.ShapeDtypeStruct((_NC * K,), jnp.float32),
        ],
        scratch_types=[
            pltpu.VMEM((2, _HALF), jnp.int32),
            pltpu.VMEM((_BPW, DIM), jnp.float32),
            pltpu.VMEM((_KPC,), jnp.float32),
            pltpu.VMEM((16 * ((_HALF + 15) // 16),), jnp.float32),
            pltpu.VMEM_SHARED((K,), jnp.float32),
            pltpu.SemaphoreType.DMA,
        ],
    )


def _epilogue_body(cnt_ref, x_ref, r7_ref, q7_ref, sse_ref,
                   qout_ref, perp_ref, com_ref):
    lvl = pl.program_id(0)
    counts = cnt_ref[0]
    avg = jnp.sum(counts, axis=0, keepdims=True) * jnp.float32(1.0 / N)
    ent = jnp.sum(avg * jnp.log(avg + 1e-10))
    perp_ref[...] = jnp.exp(-ent)[None, None, None]

    @pl.when(lvl == 0)
    def _final():
        x = x_ref[...]
        r8 = r7_ref[...] - q7_ref[...]
        qsum = x - r8
        qout_ref[...] = x + (qsum - x)
        sse8 = jnp.sum(r8 * r8)
        tot = (jnp.sum(sse_ref[...]) + sse8) * jnp.float32(1.0 / (N * DIM))
        com_ref[...] = (tot * jnp.float32(COMMIT_W))[None, None]


_epilogue = pl.pallas_call(
    _epilogue_body,
    grid=(L,),
    in_specs=[
        pl.BlockSpec((1, _NC, K), lambda l: (l, 0, 0)),
        pl.BlockSpec((N, DIM), lambda l: (0, 0)),
        pl.BlockSpec((N, DIM), lambda l: (0, 0)),
        pl.BlockSpec((N, DIM), lambda l: (0, 0)),
        pl.BlockSpec((L, 1), lambda l: (0, 0)),
    ],
    out_specs=[
        pl.BlockSpec((N, DIM), lambda l: (0, 0)),
        pl.BlockSpec((1, 1, 1), lambda l: (l, 0, 0)),
        pl.BlockSpec((1, 1), lambda l: (0, 0)),
    ],
    out_shape=[
        jax.ShapeDtypeStruct((N, DIM), jnp.float32),
        jax.ShapeDtypeStruct((L, 1, 1), jnp.float32),
        jax.ShapeDtypeStruct((1, 1), jnp.float32),
    ],
    compiler_params=pltpu.CompilerParams(
        dimension_semantics=("arbitrary",)),
)


def _run_levels(x_flat, codebooks, b2_all):
    idx_list, cnt_list, sse_list = [], [], []
    res = None
    quant = None
    for lvl in range(L):
        cb = codebooks[lvl]
        b2 = b2_all[lvl]
        if lvl == 0:
            res, idx, _ = _level_first(x_flat, cb, b2)
        else:
            res, idx, sse = _level_next(res, quant, cb, b2)
            sse_list.append(sse)
        idx_list.append(idx)
        quant, counts = _get_gather_count()(cb, idx.reshape(N))
        cnt_list.append(counts)
    return idx_list, cnt_list, sse_list, res, quant


def kernel(x, codebooks):
    x_flat = x.reshape(N, DIM)
    b2_all = _b2_call(codebooks)
    idx_list, cnt_list, sse_list, r7, q7 = _run_levels(
        x_flat, codebooks, b2_all)

    counts_part = jnp.stack([c.reshape(_NC, K) for c in cnt_list])
    sse_in = jnp.concatenate(
        sse_list + [jnp.zeros((1, 1), jnp.float32)], axis=0)

    qout_flat, perp, com = _epilogue(counts_part, x_flat, r7, q7, sse_in)

    indices = jnp.concatenate(idx_list, axis=1).reshape(B, S, L)
    quantized_out = qout_flat.reshape(B, S, DIM)
    perplexities = perp.reshape(L)
    total_commit = com.reshape(())
    dead = jnp.zeros((L,), jnp.int32)
    return quantized_out, indices, total_commit, perplexities, dead


# speedup vs baseline: 1.9259x; 1.5154x over previous
import functools

import jax
import jax.numpy as jnp
from jax import lax
from jax.experimental import pallas as pl
from jax.experimental.pallas import tpu as pltpu
from jax.experimental.pallas import tpu_sc as plsc

B, S, DIM = 8, 576, 256
N = B * S
K = 8192
L = 8
COMMIT_W = 0.25

N_TILE = 512
NT = N // N_TILE


def _dist_argmin_body(has_q, *refs):
    if has_q:
        (r_ref, q_ref, cb_ref, b2_ref, res_ref, idx_ref, sse_ref) = refs
    else:
        (r_ref, cb_ref, b2_ref, res_ref, idx_ref, sse_ref) = refs
    n = pl.program_id(0)

    r = r_ref[...]
    if has_q:
        r = r - q_ref[...]
    res_ref[...] = r
    rsq = r * r
    a2 = jnp.sum(rsq, axis=1, keepdims=True)
    part = jnp.sum(rsq)[None, None]
    sse_ref[...] = jnp.where(n == 0, part, sse_ref[...] + part)

    t = lax.dot_general(r, cb_ref[...], (((1,), (1,)), ((), ())),
                        precision=lax.Precision.DEFAULT,
                        preferred_element_type=jnp.float32)
    d2 = (a2 + b2_ref[...]) - 2.0 * t
    tmin = jnp.min(d2, axis=1, keepdims=True)
    ii = lax.broadcasted_iota(jnp.int32, d2.shape, 1)
    idx_ref[...] = jnp.min(jnp.where(d2 == tmin, ii, jnp.int32(K)),
                           axis=1, keepdims=True)


def _make_level_call(has_q, interpret=False):
    in_specs = [pl.BlockSpec((N_TILE, DIM), lambda n: (n, 0))]
    if has_q:
        in_specs.append(pl.BlockSpec((N_TILE, DIM), lambda n: (n, 0)))
    in_specs += [
        pl.BlockSpec((K, DIM), lambda n: (0, 0)),
        pl.BlockSpec((1, K), lambda n: (0, 0)),
    ]
    return pl.pallas_call(
        functools.partial(_dist_argmin_body, has_q),
        grid=(NT,),
        in_specs=in_specs,
        out_specs=[
            pl.BlockSpec((N_TILE, DIM), lambda n: (n, 0)),
            pl.BlockSpec((N_TILE, 1), lambda n: (n, 0)),
            pl.BlockSpec((1, 1), lambda n: (0, 0)),
        ],
        out_shape=[
            jax.ShapeDtypeStruct((N, DIM), jnp.float32),
            jax.ShapeDtypeStruct((N, 1), jnp.int32),
            jax.ShapeDtypeStruct((1, 1), jnp.float32),
        ],
        compiler_params=pltpu.CompilerParams(
            dimension_semantics=("arbitrary",)),
        interpret=interpret,
    )


_level_first = _make_level_call(False)
_level_next = _make_level_call(True)


def _b2_body(cb_ref, out_ref):
    cb = cb_ref[0]
    cbsq = cb * cb
    ones = jnp.ones((1, DIM), jnp.float32)
    out_ref[0] = lax.dot_general(ones, cbsq, (((1,), (1,)), ((), ())),
                                 precision=lax.Precision.HIGHEST,
                                 preferred_element_type=jnp.float32)


_b2_call = pl.pallas_call(
    _b2_body,
    grid=(L,),
    in_specs=[pl.BlockSpec((1, K, DIM), lambda l: (l, 0, 0))],
    out_specs=pl.BlockSpec((1, 1, K), lambda l: (l, 0, 0)),
    out_shape=jax.ShapeDtypeStruct((L, 1, K), jnp.float32),
    compiler_params=pltpu.CompilerParams(
        dimension_semantics=("arbitrary",)),
)


_NC, _NS = 2, 16
_NW = _NC * _NS
_BPW = N // _NW
_HALF = _BPW // 2
_KPC = K // _NS


def _sc_gather_count(cb_hbm, idx_hbm, out_hbm, counts_hbm,
                     idx_v, rows_v, stage_v, ones_v, cnt_sh, sem):
    c = lax.axis_index("c")
    s = lax.axis_index("s")
    wid = s * _NC + c
    base = wid * _BPW
    for j in range(2):
        pltpu.sync_copy(idx_hbm.at[pl.ds(base + j * _HALF, _HALF)],
                        idx_v.at[j])
    cps = [pltpu.async_copy(cb_hbm.at[idx_v.at[j]],
                            rows_v.at[pl.ds(j * _HALF, _HALF)], sem)
           for j in range(2)]
    for cp in cps:
        cp.wait()
    pltpu.sync_copy(rows_v, out_hbm.at[pl.ds(base, _BPW)])

    def _zero(i, _):
        stage_v[pl.ds(i * 16, 16)] = jnp.zeros((16,), jnp.float32)
        return 0

    lax.fori_loop(0, _KPC // 16, _zero, 0)

    def _one(i, _):
        ones_v[pl.ds(i * 16, 16)] = jnp.ones((16,), jnp.float32)
        return 0

    lax.fori_loop(0, (_HALF + 15) // 16, _one, 0)

    pltpu.sync_copy(stage_v, cnt_sh.at[pl.ds(s * _KPC, _KPC)])
    plsc.subcore_barrier()
    for j in range(2):
        pltpu.sync_copy(ones_v.at[pl.ds(0, _HALF)],
                        cnt_sh.at[idx_v.at[j]], add=True)
    plsc.subcore_barrier()
    pltpu.sync_copy(cnt_sh.at[pl.ds(s * _KPC, _KPC)], stage_v)
    pltpu.sync_copy(stage_v, counts_hbm.at[pl.ds(c * K + s * _KPC, _KPC)])


@functools.cache
def _get_gather_count():
    return pl.kernel(
        _sc_gather_count,
        mesh=plsc.VectorSubcoreMesh(core_axis_name="c",
                                    subcore_axis_name="s"),
        out_type=[
            jax.ShapeDtypeStruct((N, DIM), jnp.float32),
            jax.ShapeDtypeStruct((_NC * K,), jnp.float32),
        ],
        scratch_types=[
            pltpu.VMEM((2, _HALF), jnp.int32),
            pltpu.VMEM((_BPW, DIM), jnp.float32),
            pltpu.VMEM((_KPC,), jnp.float32),
            pltpu.VMEM((16 * ((_HALF + 15) // 16),), jnp.float32),
            pltpu.VMEM_SHARED((K,), jnp.float32),
            pltpu.SemaphoreType.DMA,
        ],
    )


def _epilogue_body(cnt_ref, x_ref, r7_ref, q7_ref, sse_ref,
                   qout_ref, perp_ref, com_ref):
    lvl = pl.program_id(0)
    counts = cnt_ref[0]
    avg = jnp.sum(counts, axis=0, keepdims=True) * jnp.float32(1.0 / N)
    ent = jnp.sum(avg * jnp.log(avg + 1e-10))
    perp_ref[...] = jnp.exp(-ent)[None, None, None]

    @pl.when(lvl == 0)
    def _final():
        x = x_ref[...]
        r8 = r7_ref[...] - q7_ref[...]
        qsum = x - r8
        qout_ref[...] = x + (qsum - x)
        sse8 = jnp.sum(r8 * r8)
        tot = (jnp.sum(sse_ref[...]) + sse8) * jnp.float32(1.0 / (N * DIM))
        com_ref[...] = (tot * jnp.float32(COMMIT_W))[None, None]


_epilogue = pl.pallas_call(
    _epilogue_body,
    grid=(L,),
    in_specs=[
        pl.BlockSpec((1, _NC, K), lambda l: (l, 0, 0)),
        pl.BlockSpec((N, DIM), lambda l: (0, 0)),
        pl.BlockSpec((N, DIM), lambda l: (0, 0)),
        pl.BlockSpec((N, DIM), lambda l: (0, 0)),
        pl.BlockSpec((L, 1), lambda l: (0, 0)),
    ],
    out_specs=[
        pl.BlockSpec((N, DIM), lambda l: (0, 0)),
        pl.BlockSpec((1, 1, 1), lambda l: (l, 0, 0)),
        pl.BlockSpec((1, 1), lambda l: (0, 0)),
    ],
    out_shape=[
        jax.ShapeDtypeStruct((N, DIM), jnp.float32),
        jax.ShapeDtypeStruct((L, 1, 1), jnp.float32),
        jax.ShapeDtypeStruct((1, 1), jnp.float32),
    ],
    compiler_params=pltpu.CompilerParams(
        dimension_semantics=("arbitrary",)),
)


def _run_levels(x_flat, codebooks, codebooks_f32, b2_all):
    idx_list, cnt_list, sse_list = [], [], []
    res = None
    quant = None
    for lvl in range(L):
        cb_l = codebooks_f32[lvl]
        b2 = b2_all[lvl]
        if lvl == 0:
            res, idx, _ = _level_first(x_flat, cb_l, b2)
        else:
            res, idx, sse = _level_next(res, quant, cb_l, b2)
            sse_list.append(sse)
        idx_list.append(idx)
        quant, counts = _get_gather_count()(codebooks[lvl], idx.reshape(N))
        cnt_list.append(counts)
    return idx_list, cnt_list, sse_list, res, quant


def kernel(x, codebooks):
    x_flat = x.reshape(N, DIM)
    b2_all = _b2_call(codebooks)
    idx_list, cnt_list, sse_list, r7, q7 = _run_levels(
        x_flat, codebooks, codebooks, b2_all)

    counts_part = jnp.stack([c.reshape(_NC, K) for c in cnt_list])
    sse_in = jnp.concatenate(
        sse_list + [jnp.zeros((1, 1), jnp.float32)], axis=0)

    qout_flat, perp, com = _epilogue(counts_part, x_flat, r7, q7, sse_in)

    indices = jnp.concatenate(idx_list, axis=1).reshape(B, S, L)
    quantized_out = qout_flat.reshape(B, S, DIM)
    perplexities = perp.reshape(L)
    total_commit = com.reshape(())
    dead = jnp.zeros((L,), jnp.int32)
    return quantized_out, indices, total_commit, perplexities, dead


# speedup vs baseline: 2.0670x; 1.0733x over previous
import functools

import jax
import jax.numpy as jnp
from jax import lax
from jax.experimental import pallas as pl
from jax.experimental.pallas import tpu as pltpu
from jax.experimental.pallas import tpu_sc as plsc

B, S, DIM = 8, 576, 256
N = B * S
K = 8192
L = 8
COMMIT_W = 0.25

N_TILE = 512
NT = N // N_TILE


K_CHUNK = 512
KC = K // K_CHUNK


def _dist_argmin_body(has_q, *refs):
    if has_q:
        (r_ref, q_ref, cb_ref, b2_ref, res_ref, idx_ref, sse_ref) = refs
    else:
        (r_ref, cb_ref, b2_ref, res_ref, idx_ref, sse_ref) = refs
    n = pl.program_id(0)

    r = r_ref[...]
    if has_q:
        r = r - q_ref[...]
    res_ref[...] = r
    rsq = r * r
    a2 = jnp.sum(rsq, axis=1, keepdims=True)
    part = jnp.sum(rsq)[None, None]
    sse_ref[...] = jnp.where(n == 0, part, sse_ref[...] + part)

    r2_bf = r + r
    cb = cb_ref[...]
    best_v = None
    best_i = None
    for j in range(KC):
        sl = slice(j * K_CHUNK, (j + 1) * K_CHUNK)
        t2 = lax.dot_general(r2_bf, cb[sl, :], (((1,), (1,)), ((), ())),
                             precision=lax.Precision.DEFAULT,
                             preferred_element_type=jnp.float32)
        d2 = (a2 + b2_ref[:, sl]) - t2
        ii = lax.broadcasted_iota(jnp.int32, d2.shape, 1) + jnp.int32(
            j * K_CHUNK)
        if j == 0:
            best_v, best_i = d2, ii
        else:
            lt = d2 < best_v
            best_v = jnp.minimum(best_v, d2)
            best_i = jnp.where(lt, ii, best_i)
    tmin = jnp.min(best_v, axis=1, keepdims=True)
    idx_ref[...] = jnp.min(jnp.where(best_v == tmin, best_i, jnp.int32(K)),
                           axis=1, keepdims=True)


def _make_level_call(has_q, interpret=False):
    in_specs = [pl.BlockSpec((N_TILE, DIM), lambda n: (n, 0))]
    if has_q:
        in_specs.append(pl.BlockSpec((N_TILE, DIM), lambda n: (n, 0)))
    in_specs += [
        pl.BlockSpec((K, DIM), lambda n: (0, 0)),
        pl.BlockSpec((1, K), lambda n: (0, 0)),
    ]
    return pl.pallas_call(
        functools.partial(_dist_argmin_body, has_q),
        grid=(NT,),
        in_specs=in_specs,
        out_specs=[
            pl.BlockSpec((N_TILE, DIM), lambda n: (n, 0)),
            pl.BlockSpec((N_TILE, 1), lambda n: (n, 0)),
            pl.BlockSpec((1, 1), lambda n: (0, 0)),
        ],
        out_shape=[
            jax.ShapeDtypeStruct((N, DIM), jnp.float32),
            jax.ShapeDtypeStruct((N, 1), jnp.int32),
            jax.ShapeDtypeStruct((1, 1), jnp.float32),
        ],
        compiler_params=pltpu.CompilerParams(
            dimension_semantics=("arbitrary",)),
        interpret=interpret,
    )


_level_first = _make_level_call(False)
_level_next = _make_level_call(True)


def _b2_body(cb_ref, out_ref):
    cb = cb_ref[0]
    cbsq = cb * cb
    ones = jnp.ones((1, DIM), jnp.float32)
    out_ref[0] = lax.dot_general(ones, cbsq, (((1,), (1,)), ((), ())),
                                 precision=lax.Precision.HIGHEST,
                                 preferred_element_type=jnp.float32)


_b2_call = pl.pallas_call(
    _b2_body,
    grid=(L,),
    in_specs=[pl.BlockSpec((1, K, DIM), lambda l: (l, 0, 0))],
    out_specs=pl.BlockSpec((1, 1, K), lambda l: (l, 0, 0)),
    out_shape=jax.ShapeDtypeStruct((L, 1, K), jnp.float32),
    compiler_params=pltpu.CompilerParams(
        dimension_semantics=("arbitrary",)),
)


_NC, _NS = 2, 16
_NW = _NC * _NS
_BPW = N // _NW
_HALF = _BPW // 2
_KPC = K // _NS


def _sc_gather_count(cb_hbm, idx_hbm, out_hbm, counts_hbm,
                     idx_v, rows_v, stage_v, ones_v, cnt_sh, sem):
    c = lax.axis_index("c")
    s = lax.axis_index("s")
    wid = s * _NC + c
    base = wid * _BPW
    for j in range(2):
        pltpu.sync_copy(idx_hbm.at[pl.ds(base + j * _HALF, _HALF)],
                        idx_v.at[j])
    cps = [pltpu.async_copy(cb_hbm.at[idx_v.at[j]],
                            rows_v.at[pl.ds(j * _HALF, _HALF)], sem)
           for j in range(2)]
    for cp in cps:
        cp.wait()
    pltpu.sync_copy(rows_v, out_hbm.at[pl.ds(base, _BPW)])

    def _zero(i, _):
        stage_v[pl.ds(i * 16, 16)] = jnp.zeros((16,), jnp.float32)
        return 0

    lax.fori_loop(0, _KPC // 16, _zero, 0)

    def _one(i, _):
        ones_v[pl.ds(i * 16, 16)] = jnp.ones((16,), jnp.float32)
        return 0

    lax.fori_loop(0, (_HALF + 15) // 16, _one, 0)

    pltpu.sync_copy(stage_v, cnt_sh.at[pl.ds(s * _KPC, _KPC)])
    plsc.subcore_barrier()
    for j in range(2):
        pltpu.sync_copy(ones_v.at[pl.ds(0, _HALF)],
                        cnt_sh.at[idx_v.at[j]], add=True)
    plsc.subcore_barrier()
    pltpu.sync_copy(cnt_sh.at[pl.ds(s * _KPC, _KPC)], stage_v)
    pltpu.sync_copy(stage_v, counts_hbm.at[pl.ds(c * K + s * _KPC, _KPC)])


@functools.cache
def _get_gather_count():
    return pl.kernel(
        _sc_gather_count,
        mesh=plsc.VectorSubcoreMesh(core_axis_name="c",
                                    subcore_axis_name="s"),
        out_type=[
            jax.ShapeDtypeStruct((N, DIM), jnp.float32),
            jax.ShapeDtypeStruct((_NC * K,), jnp.float32),
        ],
        scratch_types=[
            pltpu.VMEM((2, _HALF), jnp.int32),
            pltpu.VMEM((_BPW, DIM), jnp.float32),
            pltpu.VMEM((_KPC,), jnp.float32),
            pltpu.VMEM((16 * ((_HALF + 15) // 16),), jnp.float32),
            pltpu.VMEM_SHARED((K,), jnp.float32),
            pltpu.SemaphoreType.DMA,
        ],
    )


def _epilogue_body(cnt_ref, x_ref, r7_ref, q7_ref, sse_ref,
                   qout_ref, perp_ref, com_ref):
    lvl = pl.program_id(0)
    counts = cnt_ref[0]
    avg = jnp.sum(counts, axis=0, keepdims=True) * jnp.float32(1.0 / N)
    ent = jnp.sum(avg * jnp.log(avg + 1e-10))
    perp_ref[...] = jnp.exp(-ent)[None, None, None]

    @pl.when(lvl == 0)
    def _final():
        x = x_ref[...]
        r8 = r7_ref[...] - q7_ref[...]
        qsum = x - r8
        qout_ref[...] = x + (qsum - x)
        sse8 = jnp.sum(r8 * r8)
        tot = (jnp.sum(sse_ref[...]) + sse8) * jnp.float32(1.0 / (N * DIM))
        com_ref[...] = (tot * jnp.float32(COMMIT_W))[None, None]


_epilogue = pl.pallas_call(
    _epilogue_body,
    grid=(L,),
    in_specs=[
        pl.BlockSpec((1, _NC, K), lambda l: (l, 0, 0)),
        pl.BlockSpec((N, DIM), lambda l: (0, 0)),
        pl.BlockSpec((N, DIM), lambda l: (0, 0)),
        pl.BlockSpec((N, DIM), lambda l: (0, 0)),
        pl.BlockSpec((L, 1), lambda l: (0, 0)),
    ],
    out_specs=[
        pl.BlockSpec((N, DIM), lambda l: (0, 0)),
        pl.BlockSpec((1, 1, 1), lambda l: (l, 0, 0)),
        pl.BlockSpec((1, 1), lambda l: (0, 0)),
    ],
    out_shape=[
        jax.ShapeDtypeStruct((N, DIM), jnp.float32),
        jax.ShapeDtypeStruct((L, 1, 1), jnp.float32),
        jax.ShapeDtypeStruct((1, 1), jnp.float32),
    ],
    compiler_params=pltpu.CompilerParams(
        dimension_semantics=("arbitrary",)),
)


def _run_levels(x_flat, codebooks, codebooks_bf, b2_all):
    idx_list, cnt_list, sse_list = [], [], []
    res = None
    quant = None
    for lvl in range(L):
        cb_l = codebooks_bf[lvl]
        b2 = b2_all[lvl]
        if lvl == 0:
            res, idx, _ = _level_first(x_flat, cb_l, b2)
        else:
            res, idx, sse = _level_next(res, quant, cb_l, b2)
            sse_list.append(sse)
        idx_list.append(idx)
        quant, counts = _get_gather_count()(codebooks[lvl], idx.reshape(N))
        cnt_list.append(counts)
    return idx_list, cnt_list, sse_list, res, quant


def kernel(x, codebooks):
    x_flat = x.reshape(N, DIM)
    b2_all = _b2_call(codebooks)
    idx_list, cnt_list, sse_list, r7, q7 = _run_levels(
        x_flat, codebooks, codebooks, b2_all)

    counts_part = jnp.stack([c.reshape(_NC, K) for c in cnt_list])
    sse_in = jnp.concatenate(
        sse_list + [jnp.zeros((1, 1), jnp.float32)], axis=0)

    qout_flat, perp, com = _epilogue(counts_part, x_flat, r7, q7, sse_in)

    indices = jnp.concatenate(idx_list, axis=1).reshape(B, S, L)
    quantized_out = qout_flat.reshape(B, S, DIM)
    perplexities = perp.reshape(L)
    total_commit = com.reshape(())
    dead = jnp.zeros((L,), jnp.int32)
    return quantized_out, indices, total_commit, perplexities, dead
